# baseline (speedup 1.0000x reference)
import functools

import jax
import jax.numpy as jnp
import numpy as np
from jax import lax
from jax.experimental import pallas as pl
from jax.experimental.pallas import tpu as pltpu
from jax.experimental.pallas import tpu_sc as plsc

PS = 16
DPR = 24
DPC = 24
SPR = 48
SPC = 48
S = 4
B = 8
M = 288
NQ = DPR * DPC

N_OUT_ROWS = B * 3 * DPR * PS * DPC
N_SRC_ROWS = S * B * 3 * SPR * PS * SPC
ROWS_PER_B = 3 * DPR * PS * DPC
SRC_ROWS_PER_IMG = 3 * SPR * PS * SPC
SEG_PER_IMG = SPR * PS * SPC

N_TILES = 32
ROWS_PER_TILE = N_OUT_ROWS // N_TILES
DMA_ROWS = 128
N_DMA = ROWS_PER_TILE // DMA_ROWS
M_VECS = M // 16
DEPTH = 12


def _build_comb():
    o = np.arange(N_OUT_ROWS, dtype=np.int64)
    c = (o // 9216) % 3
    t = o % 9216
    ri2 = t // 192
    ci2 = (t // 64) % 3
    wy = (t // 8) % 8
    sx = t % 8
    y = ri2 * 8 + wy
    r = y % PS
    q = (y // PS) * DPC + (ci2 * 8 + sx)
    offs = c * SEG_PER_IMG + (r // 8) * 384 + (r % 8) * 8
    return (offs * 1024 + q).astype(np.int32)


_COMB = _build_comb()


@functools.cache
def _make_sc_remap():
    mesh = plsc.VectorSubcoreMesh(core_axis_name="c", subcore_axis_name="s")

    @functools.partial(
        pl.kernel,
        out_type=jax.ShapeDtypeStruct((N_OUT_ROWS, PS), jnp.float32),
        mesh=mesh,
        scratch_types=[
        pltpu.VMEM((4 * M,), jnp.int32),
        pltpu.VMEM((NQ,), jnp.int32),
        pltpu.VMEM((ROWS_PER_TILE,), jnp.int32),
        pltpu.VMEM((ROWS_PER_TILE,), jnp.int32),
            pltpu.VMEM((ROWS_PER_TILE, PS), jnp.float32),
            pltpu.SemaphoreType.DMA,
            pltpu.SemaphoreType.DMA,
        ],
        compiler_params=pltpu.CompilerParams(use_tc_tiling_on_sc=False, needs_layout_passes=False),
    )
    def _sc_remap(src_hbm, meta_hbm, comb_hbm, out_hbm,
                  meta_v, a_v, comb_v, idx_v, buf_v, gsem, csem):
        wid = lax.axis_index("s") * 2 + lax.axis_index("c")
        b = wid // 4
        base = wid * ROWS_PER_TILE

        cp_meta = pltpu.async_copy(
            meta_hbm.at[pl.ds(b * 4 * M, 4 * M)], meta_v, csem)
        cp_comb = pltpu.async_copy(
            comb_hbm.at[pl.ds(base, ROWS_PER_TILE)], comb_v, csem)
        cp_meta.wait()

        def splat(x):
            return jnp.full((16,), x, jnp.int32)

        c_img = splat(B * SRC_ROWS_PER_IMG)
        c_48 = splat(SPC)
        c_768 = splat(768)
        c_8 = splat(8)
        c_64 = splat(64)
        c_boff = lax.broadcast(b * SRC_ROWS_PER_IMG, (16,))
        for i in range(M_VECS):
            siv = meta_v[pl.ds(i * 16, 16)]
            spv = meta_v[pl.ds(M + i * 16, 16)]
            d0v = meta_v[pl.ds(2 * M + i * 16, 16)]
            d1v = meta_v[pl.ds(3 * M + i * 16, 16)]
            spc8 = lax.div(lax.rem(spv, c_48), c_8)
            val = (siv * c_img + c_boff
                   + lax.div(spv, c_48) * c_768
                   + spc8 * c_64 + lax.rem(spv, c_8))
            plsc.store_scatter(a_v, [d0v], val)
            plsc.store_scatter(a_v, [d1v], val)
        cp_comb.wait()

        c_mask = splat(1023)
        c_shift = splat(10)

        def step(i, carry):
            cv = comb_v[pl.ds(i * 16, 16)]
            qv = lax.bitwise_and(cv, c_mask)
            ov = lax.shift_right_logical(cv, c_shift)
            av = plsc.load_gather(a_v, [qv])
            idx_v[pl.ds(i * 16, 16)] = av + ov
            return carry

        lax.fori_loop(0, ROWS_PER_TILE // 16, step, 0)
        pltpu.async_copy(src_hbm.at[idx_v], buf_v, gsem).wait()
        pltpu.sync_copy(buf_v, out_hbm.at[pl.ds(base, ROWS_PER_TILE)])

    return _sc_remap


def kernel(src_images, mappings):
    src_table = (
        src_images.reshape(S, B, 3, 96, 8, 6, 128)
        .transpose(0, 1, 2, 3, 5, 4, 6)
        .reshape(N_SRC_ROWS, PS))
    mp = mappings.astype(jnp.int32)
    meta = jnp.stack(
        [mp[:, :, 0, 0], mp[:, :, 0, 1], mp[:, :, 1, 0], mp[:, :, 1, 1]],
        axis=1).reshape(-1)
    out = _make_sc_remap()(src_table, meta, jnp.asarray(_COMB))
    return (out.reshape(B, 3, 48, 3, 8, 128)
            .transpose(0, 1, 2, 4, 3, 5)
            .reshape(B, 3, DPR * PS, DPC * PS))


# speedup vs baseline: 1.0517x; 1.0517x over previous
import functools

import jax
import jax.numpy as jnp
import numpy as np
from jax import lax
from jax.experimental import pallas as pl
from jax.experimental.pallas import tpu as pltpu
from jax.experimental.pallas import tpu_sc as plsc

PS = 16
DPR = 24
DPC = 24
SPR = 48
SPC = 48
S = 4
B = 8
M = 288
NQ = DPR * DPC

N_OUT_ROWS = B * 3 * DPR * PS * DPC
N_SRC_ROWS = S * B * 3 * SPR * PS * SPC
ROWS_PER_B = 3 * DPR * PS * DPC
SRC_ROWS_PER_IMG = 3 * SPR * PS * SPC
SEG_PER_IMG = SPR * PS * SPC

N_TILES = 32
ROWS_PER_TILE = N_OUT_ROWS // N_TILES
DMA_ROWS = 128
N_DMA = ROWS_PER_TILE // DMA_ROWS
M_VECS = M // 16
DEPTH = 12


def _build_comb():
    o = np.arange(N_OUT_ROWS, dtype=np.int64)
    c = (o // 9216) % 3
    t = o % 9216
    ri2 = t // 192
    ci2 = (t // 64) % 3
    wy = (t // 8) % 8
    sx = t % 8
    y = ri2 * 8 + wy
    r = y % PS
    q = (y // PS) * DPC + (ci2 * 8 + sx)
    offs = c * SEG_PER_IMG + (r // 8) * 384 + (r % 8) * 8
    return (offs * 1024 + q).astype(np.int32)


_COMB = _build_comb()


@functools.cache
def _make_sc_remap():
    mesh = plsc.VectorSubcoreMesh(core_axis_name="c", subcore_axis_name="s")

    @functools.partial(
        pl.kernel,
        out_type=jax.ShapeDtypeStruct((N_OUT_ROWS, PS), jnp.float32),
        mesh=mesh,
        scratch_types=[
        pltpu.VMEM((4 * M,), jnp.int32),
        pltpu.VMEM((NQ,), jnp.int32),
        pltpu.VMEM((ROWS_PER_TILE,), jnp.int32),
        pltpu.VMEM((N_DMA, DMA_ROWS), jnp.int32),
            pltpu.VMEM((ROWS_PER_TILE, PS), jnp.float32),
            pltpu.SemaphoreType.DMA,
            pltpu.SemaphoreType.DMA,
        ],
        compiler_params=pltpu.CompilerParams(use_tc_tiling_on_sc=False, needs_layout_passes=False),
    )
    def _sc_remap(src_hbm, meta_hbm, comb_hbm, out_hbm,
                  meta_v, a_v, comb_v, idx_v, buf_v, gsem, csem):
        wid = lax.axis_index("s") * 2 + lax.axis_index("c")
        b = wid // 4
        base = wid * ROWS_PER_TILE

        cp_meta = pltpu.async_copy(
            meta_hbm.at[pl.ds(b * 4 * M, 4 * M)], meta_v, csem)
        cp_comb = pltpu.async_copy(
            comb_hbm.at[pl.ds(base, ROWS_PER_TILE)], comb_v, csem)
        cp_meta.wait()

        def splat(x):
            return jnp.full((16,), x, jnp.int32)

        c_img = splat(B * SRC_ROWS_PER_IMG)
        c_48 = splat(SPC)
        c_768 = splat(768)
        c_8 = splat(8)
        c_64 = splat(64)
        c_boff = lax.broadcast(b * SRC_ROWS_PER_IMG, (16,))
        for i in range(M_VECS):
            siv = meta_v[pl.ds(i * 16, 16)]
            spv = meta_v[pl.ds(M + i * 16, 16)]
            d0v = meta_v[pl.ds(2 * M + i * 16, 16)]
            d1v = meta_v[pl.ds(3 * M + i * 16, 16)]
            spc8 = lax.div(lax.rem(spv, c_48), c_8)
            val = (siv * c_img + c_boff
                   + lax.div(spv, c_48) * c_768
                   + spc8 * c_64 + lax.rem(spv, c_8))
            plsc.store_scatter(a_v, [d0v], val)
            plsc.store_scatter(a_v, [d1v], val)
        cp_comb.wait()

        c_mask = splat(1023)
        c_shift = splat(10)

        def step(j, carry):
            for rr in range(DMA_ROWS // 16):
                cv = comb_v[pl.ds(j * DMA_ROWS + rr * 16, 16)]
                qv = lax.bitwise_and(cv, c_mask)
                ov = lax.shift_right_logical(cv, c_shift)
                av = plsc.load_gather(a_v, [qv])
                idx_v[j, pl.ds(rr * 16, 16)] = av + ov
            pltpu.async_copy(
                src_hbm.at[idx_v.at[j]],
                buf_v.at[pl.ds(j * DMA_ROWS, DMA_ROWS)], gsem)

            @pl.when(j >= DEPTH)
            def _():
                jw = j - DEPTH
                pltpu.make_async_copy(
                    src_hbm.at[idx_v.at[jw]],
                    buf_v.at[pl.ds(jw * DMA_ROWS, DMA_ROWS)], gsem).wait()
                pltpu.async_copy(
                    buf_v.at[pl.ds(jw * DMA_ROWS, DMA_ROWS)],
                    out_hbm.at[pl.ds(base + jw * DMA_ROWS, DMA_ROWS)], csem)
            return carry

        lax.fori_loop(0, N_DMA, step, 0)
        for jw in range(N_DMA - DEPTH, N_DMA):
            pltpu.make_async_copy(
                src_hbm.at[idx_v.at[jw]],
                buf_v.at[pl.ds(jw * DMA_ROWS, DMA_ROWS)], gsem).wait()
            pltpu.async_copy(
                buf_v.at[pl.ds(jw * DMA_ROWS, DMA_ROWS)],
                out_hbm.at[pl.ds(base + jw * DMA_ROWS, DMA_ROWS)], csem)

        def drain(j, carry):
            pltpu.make_async_copy(
                buf_v.at[pl.ds(j * DMA_ROWS, DMA_ROWS)],
                out_hbm.at[pl.ds(base + j * DMA_ROWS, DMA_ROWS)], csem).wait()
            return carry

        lax.fori_loop(0, N_DMA, drain, 0)

    return _sc_remap


def kernel(src_images, mappings):
    src_table = (
        src_images.reshape(S, B, 3, 96, 8, 6, 128)
        .transpose(0, 1, 2, 3, 5, 4, 6)
        .reshape(N_SRC_ROWS, PS))
    mp = mappings.astype(jnp.int32)
    meta = jnp.stack(
        [mp[:, :, 0, 0], mp[:, :, 0, 1], mp[:, :, 1, 0], mp[:, :, 1, 1]],
        axis=1).reshape(-1)
    out = _make_sc_remap()(src_table, meta, jnp.asarray(_COMB))
    return (out.reshape(B, 3, 48, 3, 8, 128)
            .transpose(0, 1, 2, 4, 3, 5)
            .reshape(B, 3, DPR * PS, DPC * PS))


# speedup vs baseline: 1.0527x; 1.0009x over previous
import functools

import jax
import jax.numpy as jnp
import numpy as np
from jax import lax
from jax.experimental import pallas as pl
from jax.experimental.pallas import tpu as pltpu
from jax.experimental.pallas import tpu_sc as plsc

PS = 16
DPR = 24
DPC = 24
SPR = 48
SPC = 48
S = 4
B = 8
M = 288
NQ = DPR * DPC

N_OUT_ROWS = B * 3 * DPR * PS * DPC
N_SRC_ROWS = S * B * 3 * SPR * PS * SPC
ROWS_PER_B = 3 * DPR * PS * DPC
SRC_ROWS_PER_IMG = 3 * SPR * PS * SPC
SEG_PER_IMG = SPR * PS * SPC

N_TILES = 32
ROWS_PER_TILE = N_OUT_ROWS // N_TILES
DMA_ROWS = 128
N_DMA = ROWS_PER_TILE // DMA_ROWS
M_VECS = M // 16
DEPTH = 27


def _build_comb():
    o = np.arange(N_OUT_ROWS, dtype=np.int64)
    c = (o // 9216) % 3
    t = o % 9216
    ri2 = t // 192
    ci2 = (t // 64) % 3
    wy = (t // 8) % 8
    sx = t % 8
    y = ri2 * 8 + wy
    r = y % PS
    q = (y // PS) * DPC + (ci2 * 8 + sx)
    offs = c * SEG_PER_IMG + (r // 8) * 384 + (r % 8) * 8
    return (offs * 1024 + q).astype(np.int32)


_COMB = _build_comb()


@functools.cache
def _make_sc_remap():
    mesh = plsc.VectorSubcoreMesh(core_axis_name="c", subcore_axis_name="s")

    @functools.partial(
        pl.kernel,
        out_type=jax.ShapeDtypeStruct((N_OUT_ROWS, PS), jnp.float32),
        mesh=mesh,
        scratch_types=[
        pltpu.VMEM((4 * M,), jnp.int32),
        pltpu.VMEM((NQ,), jnp.int32),
        pltpu.VMEM((ROWS_PER_TILE,), jnp.int32),
        pltpu.VMEM((N_DMA, DMA_ROWS), jnp.int32),
            pltpu.VMEM((ROWS_PER_TILE, PS), jnp.float32),
            pltpu.SemaphoreType.DMA,
            pltpu.SemaphoreType.DMA,
        ],
        compiler_params=pltpu.CompilerParams(use_tc_tiling_on_sc=False, needs_layout_passes=False),
    )
    def _sc_remap(src_hbm, meta_hbm, comb_hbm, out_hbm,
                  meta_v, a_v, comb_v, idx_v, buf_v, gsem, csem):
        wid = lax.axis_index("s") * 2 + lax.axis_index("c")
        b = wid // 4
        base = wid * ROWS_PER_TILE

        cp_meta = pltpu.async_copy(
            meta_hbm.at[pl.ds(b * 4 * M, 4 * M)], meta_v, csem)
        cp_comb = pltpu.async_copy(
            comb_hbm.at[pl.ds(base, ROWS_PER_TILE)], comb_v, csem)
        cp_meta.wait()

        def splat(x):
            return jnp.full((16,), x, jnp.int32)

        c_img = splat(B * SRC_ROWS_PER_IMG)
        c_48 = splat(SPC)
        c_768 = splat(768)
        c_8 = splat(8)
        c_64 = splat(64)
        c_boff = lax.broadcast(b * SRC_ROWS_PER_IMG, (16,))
        for i in range(M_VECS):
            siv = meta_v[pl.ds(i * 16, 16)]
            spv = meta_v[pl.ds(M + i * 16, 16)]
            d0v = meta_v[pl.ds(2 * M + i * 16, 16)]
            d1v = meta_v[pl.ds(3 * M + i * 16, 16)]
            spc8 = lax.div(lax.rem(spv, c_48), c_8)
            val = (siv * c_img + c_boff
                   + lax.div(spv, c_48) * c_768
                   + spc8 * c_64 + lax.rem(spv, c_8))
            plsc.store_scatter(a_v, [d0v], val)
            plsc.store_scatter(a_v, [d1v], val)
        cp_comb.wait()

        c_mask = splat(1023)
        c_shift = splat(10)

        def step(j, carry):
            for rr in range(DMA_ROWS // 16):
                cv = comb_v[pl.ds(j * DMA_ROWS + rr * 16, 16)]
                qv = lax.bitwise_and(cv, c_mask)
                ov = lax.shift_right_logical(cv, c_shift)
                av = plsc.load_gather(a_v, [qv])
                idx_v[j, pl.ds(rr * 16, 16)] = av + ov
            pltpu.async_copy(
                src_hbm.at[idx_v.at[j]],
                buf_v.at[pl.ds(j * DMA_ROWS, DMA_ROWS)], gsem)

            @pl.when(j >= DEPTH)
            def _():
                jw = j - DEPTH
                pltpu.make_async_copy(
                    src_hbm.at[idx_v.at[jw]],
                    buf_v.at[pl.ds(jw * DMA_ROWS, DMA_ROWS)], gsem).wait()
                pltpu.async_copy(
                    buf_v.at[pl.ds(jw * DMA_ROWS, DMA_ROWS)],
                    out_hbm.at[pl.ds(base + jw * DMA_ROWS, DMA_ROWS)], csem)
            return carry

        lax.fori_loop(0, N_DMA, step, 0)
        for jw in range(N_DMA - DEPTH, N_DMA):
            pltpu.make_async_copy(
                src_hbm.at[idx_v.at[jw]],
                buf_v.at[pl.ds(jw * DMA_ROWS, DMA_ROWS)], gsem).wait()
            pltpu.async_copy(
                buf_v.at[pl.ds(jw * DMA_ROWS, DMA_ROWS)],
                out_hbm.at[pl.ds(base + jw * DMA_ROWS, DMA_ROWS)], csem)

        def drain(j, carry):
            pltpu.make_async_copy(
                buf_v.at[pl.ds(j * DMA_ROWS, DMA_ROWS)],
                out_hbm.at[pl.ds(base + j * DMA_ROWS, DMA_ROWS)], csem).wait()
            return carry

        lax.fori_loop(0, N_DMA, drain, 0)

    return _sc_remap


def kernel(src_images, mappings):
    src_table = (
        src_images.reshape(S, B, 3, 96, 8, 6, 128)
        .transpose(0, 1, 2, 3, 5, 4, 6)
        .reshape(N_SRC_ROWS, PS))
    mp = mappings.astype(jnp.int32)
    meta = jnp.stack(
        [mp[:, :, 0, 0], mp[:, :, 0, 1], mp[:, :, 1, 0], mp[:, :, 1, 1]],
        axis=1).reshape(-1)
    out = _make_sc_remap()(src_table, meta, jnp.asarray(_COMB))
    return (out.reshape(B, 3, 48, 3, 8, 128)
            .transpose(0, 1, 2, 4, 3, 5)
            .reshape(B, 3, DPR * PS, DPC * PS))
